# Initial kernel scaffold; baseline (speedup 1.0000x reference)
import jax
import jax.numpy as jnp
from jax.experimental import pallas as pl


def kernel(x, dims, thresholds, lut, S, T):
    raise NotImplementedError("write your pallas kernel here")


# speedup vs baseline: 2.6281x; 2.6281x over previous
import jax
import jax.numpy as jnp
from jax.experimental import pallas as pl

C = 64
K = 16
DEPTH = 4
N_TOKENS = 4096
IN_FEATURES = 1024
OUT_FEATURES = 1024

BN = 512

_LVL = [0, 1, 1, 2, 2, 2, 2, 3, 3, 3, 3, 3, 3, 3, 3]


def _fused_kernel(dims_ref, thr_ref, x_ref, lut_ref, out_ref):
    gi = jax.lax.broadcasted_iota(jnp.int32, (IN_FEATURES, C * DEPTH), 0)
    G = jnp.where(gi == dims_ref[...], 1.0, 0.0).astype(jnp.bfloat16)
    chosen = jnp.dot(x_ref[...], G, preferred_element_type=jnp.float32)

    g = []
    for j in range(15):
        v = chosen[:, _LVL[j] * C:(_LVL[j] + 1) * C]
        s = v - thr_ref[j:j + 1, :]
        g.append(jnp.where(s > 0, 1.0, jnp.where(s < 0, -1.0, 0.0)))

    m = [None] * 15
    b = [None] * 15
    for j in range(14, 6, -1):
        b[j] = jnp.where(g[j] > 0, 1.0, 0.0)
        m[j] = jnp.abs(g[j])
    for j in range(6, -1, -1):
        L = m[2 * j + 1] - g[j]
        R = m[2 * j + 2] + g[j]
        b[j] = jnp.where(R > L, 1.0, 0.0)
        m[j] = jnp.maximum(L, R)

    d0 = b[0]
    d1 = d0 * b[2] + (1.0 - d0) * b[1]
    d2l = d1 * b[4] + (1.0 - d1) * b[3]
    d2r = d1 * b[6] + (1.0 - d1) * b[5]
    d2 = d0 * d2r + (1.0 - d0) * d2l
    d3ll = d2 * b[8] + (1.0 - d2) * b[7]
    d3lr = d2 * b[10] + (1.0 - d2) * b[9]
    d3rl = d2 * b[12] + (1.0 - d2) * b[11]
    d3rr = d2 * b[14] + (1.0 - d2) * b[13]
    d3l = d1 * d3lr + (1.0 - d1) * d3ll
    d3r = d1 * d3rr + (1.0 - d1) * d3rl
    d3 = d0 * d3r + (1.0 - d0) * d3l
    idx = d0 * 8.0 + d1 * 4.0 + d2 * 2.0 + d3

    enc = jnp.concatenate(
        [jnp.where(idx == float(k), 1.0, 0.0) for k in range(K)], axis=1)
    out_ref[...] = jnp.dot(enc.astype(jnp.bfloat16), lut_ref[...],
                           preferred_element_type=jnp.float32)


@jax.jit
def _run(x, dims, thresholds, lut):
    dims_lm = dims.astype(jnp.int32).reshape(C, DEPTH).T.reshape(1, C * DEPTH)
    thr_r = thresholds.reshape(C, K - 1).T
    lutK = jnp.transpose(lut, (2, 1, 0)).reshape(K * C, OUT_FEATURES)
    lutK = lutK.astype(jnp.bfloat16)
    x = x.astype(jnp.bfloat16)
    return pl.pallas_call(
        _fused_kernel,
        grid=(N_TOKENS // BN,),
        in_specs=[
            pl.BlockSpec((1, C * DEPTH), lambda i: (0, 0)),
            pl.BlockSpec((K - 1, C), lambda i: (0, 0)),
            pl.BlockSpec((BN, IN_FEATURES), lambda i: (i, 0)),
            pl.BlockSpec((K * C, OUT_FEATURES), lambda i: (0, 0)),
        ],
        out_specs=pl.BlockSpec((BN, OUT_FEATURES), lambda i: (i, 0)),
        out_shape=jax.ShapeDtypeStruct((N_TOKENS, OUT_FEATURES), jnp.float32),
    )(dims_lm, thr_r, x, lutK)


def kernel(x, dims, thresholds, lut, S, T):
    return _run(x, dims, thresholds, lut)


# speedup vs baseline: 3.9013x; 1.4844x over previous
import jax
import jax.numpy as jnp
from jax.experimental import pallas as pl
from jax.experimental.pallas import tpu as pltpu

C = 64
K = 16
DEPTH = 4
N_TOKENS = 4096
IN_FEATURES = 1024
OUT_FEATURES = 1024

BN = 512

_LVL = [0, 1, 1, 2, 2, 2, 2, 3, 3, 3, 3, 3, 3, 3, 3]


def _fused_kernel(dims_ref, thr_ref, x_ref, lut_ref, out_ref, g_scr):
    @pl.when(pl.program_id(0) == 0)
    def _():
        gi = jax.lax.broadcasted_iota(jnp.int32, (IN_FEATURES, C * DEPTH), 0)
        g_scr[...] = jnp.where(gi == dims_ref[...], 1.0, 0.0).astype(jnp.bfloat16)

    chosen = jnp.dot(x_ref[...].astype(jnp.bfloat16), g_scr[...],
                     preferred_element_type=jnp.float32)

    b = []
    for j in range(15):
        v = chosen[:, _LVL[j] * C:(_LVL[j] + 1) * C]
        b.append(jnp.where(v > thr_ref[j:j + 1, :], 1.0, 0.0))

    d0 = b[0]
    d1 = d0 * b[2] + (1.0 - d0) * b[1]
    d2l = d1 * b[4] + (1.0 - d1) * b[3]
    d2r = d1 * b[6] + (1.0 - d1) * b[5]
    d2 = d0 * d2r + (1.0 - d0) * d2l
    d3ll = d2 * b[8] + (1.0 - d2) * b[7]
    d3lr = d2 * b[10] + (1.0 - d2) * b[9]
    d3rl = d2 * b[12] + (1.0 - d2) * b[11]
    d3rr = d2 * b[14] + (1.0 - d2) * b[13]
    d3l = d1 * d3lr + (1.0 - d1) * d3ll
    d3r = d1 * d3rr + (1.0 - d1) * d3rl
    d3 = d0 * d3r + (1.0 - d0) * d3l
    idx = d0 * 8.0 + d1 * 4.0 + d2 * 2.0 + d3

    enc = jnp.concatenate(
        [jnp.where(idx == float(k), 1.0, 0.0) for k in range(K)], axis=1)
    out_ref[...] = jnp.dot(enc.astype(jnp.bfloat16), lut_ref[...],
                           preferred_element_type=jnp.float32)


@jax.jit
def _run(x, dims, thresholds, lut):
    dims_lm = dims.astype(jnp.int32).reshape(C, DEPTH).T.reshape(1, C * DEPTH)
    thr_r = thresholds.reshape(C, K - 1).T
    lutK = jnp.transpose(lut, (2, 1, 0)).reshape(K * C, OUT_FEATURES)
    lutK = lutK.astype(jnp.bfloat16)
    return pl.pallas_call(
        _fused_kernel,
        grid=(N_TOKENS // BN,),
        in_specs=[
            pl.BlockSpec((1, C * DEPTH), lambda i: (0, 0)),
            pl.BlockSpec((K - 1, C), lambda i: (0, 0)),
            pl.BlockSpec((BN, IN_FEATURES), lambda i: (i, 0)),
            pl.BlockSpec((K * C, OUT_FEATURES), lambda i: (0, 0)),
        ],
        out_specs=pl.BlockSpec((BN, OUT_FEATURES), lambda i: (i, 0)),
        out_shape=jax.ShapeDtypeStruct((N_TOKENS, OUT_FEATURES), jnp.float32),
        scratch_shapes=[pltpu.VMEM((IN_FEATURES, C * DEPTH), jnp.bfloat16)],
    )(dims_lm, thr_r, x, lutK)


def kernel(x, dims, thresholds, lut, S, T):
    return _run(x, dims, thresholds, lut)


# speedup vs baseline: 4.1289x; 1.0583x over previous
import jax
import jax.numpy as jnp
from jax.experimental import pallas as pl
from jax.experimental.pallas import tpu as pltpu

C = 64
K = 16
DEPTH = 4
N_TOKENS = 4096
IN_FEATURES = 1024
OUT_FEATURES = 1024

BN = 512

_LVL = [0, 1, 1, 2, 2, 2, 2, 3, 3, 3, 3, 3, 3, 3, 3]


def _fused_kernel(dims_ref, thr_ref, x_ref, lut_ref, out_ref, g_scr):
    @pl.when(pl.program_id(0) == 0)
    def _():
        gi = jax.lax.broadcasted_iota(jnp.int32, (IN_FEATURES, C * DEPTH), 0)
        g_scr[...] = jnp.where(gi == dims_ref[...], 1.0, 0.0).astype(jnp.bfloat16)

    chosen = jnp.dot(x_ref[...].astype(jnp.bfloat16), g_scr[...],
                     preferred_element_type=jnp.float32)
    chosenT = chosen.T

    b = []
    for j in range(15):
        v = chosenT[_LVL[j] * C:(_LVL[j] + 1) * C, :]
        b.append(jnp.where(v > thr_ref[:, j:j + 1], 1.0, 0.0))

    d0 = b[0]
    d1 = d0 * b[2] + (1.0 - d0) * b[1]
    d2l = d1 * b[4] + (1.0 - d1) * b[3]
    d2r = d1 * b[6] + (1.0 - d1) * b[5]
    d2 = d0 * d2r + (1.0 - d0) * d2l
    d3ll = d2 * b[8] + (1.0 - d2) * b[7]
    d3lr = d2 * b[10] + (1.0 - d2) * b[9]
    d3rl = d2 * b[12] + (1.0 - d2) * b[11]
    d3rr = d2 * b[14] + (1.0 - d2) * b[13]
    d3l = d1 * d3lr + (1.0 - d1) * d3ll
    d3r = d1 * d3rr + (1.0 - d1) * d3rl
    d3 = d0 * d3r + (1.0 - d0) * d3l
    idx = d0 * 8.0 + d1 * 4.0 + d2 * 2.0 + d3

    encT = jnp.concatenate(
        [jnp.where(idx == float(k), 1.0, 0.0) for k in range(K)], axis=0)
    out_ref[...] = jax.lax.dot_general(
        encT.astype(jnp.bfloat16), lut_ref[...],
        dimension_numbers=(((0,), (0,)), ((), ())),
        preferred_element_type=jnp.float32)


@jax.jit
def _run(x, dims, thresholds, lut):
    dims_lm = dims.astype(jnp.int32).reshape(C, DEPTH).T.reshape(1, C * DEPTH)
    thr_r = thresholds.reshape(C, K - 1)
    lutK = jnp.transpose(lut, (2, 1, 0)).reshape(K * C, OUT_FEATURES)
    lutK = lutK.astype(jnp.bfloat16)
    return pl.pallas_call(
        _fused_kernel,
        grid=(N_TOKENS // BN,),
        in_specs=[
            pl.BlockSpec((1, C * DEPTH), lambda i: (0, 0)),
            pl.BlockSpec((C, K - 1), lambda i: (0, 0)),
            pl.BlockSpec((BN, IN_FEATURES), lambda i: (i, 0)),
            pl.BlockSpec((K * C, OUT_FEATURES), lambda i: (0, 0)),
        ],
        out_specs=pl.BlockSpec((BN, OUT_FEATURES), lambda i: (i, 0)),
        out_shape=jax.ShapeDtypeStruct((N_TOKENS, OUT_FEATURES), jnp.float32),
        scratch_shapes=[pltpu.VMEM((IN_FEATURES, C * DEPTH), jnp.bfloat16)],
    )(dims_lm, thr_r, x, lutK)


def kernel(x, dims, thresholds, lut, S, T):
    return _run(x, dims, thresholds, lut)


# speedup vs baseline: 4.3797x; 1.0607x over previous
import jax
import jax.numpy as jnp
from jax.experimental import pallas as pl
from jax.experimental.pallas import tpu as pltpu

C = 64
K = 16
DEPTH = 4
N_TOKENS = 4096
IN_FEATURES = 1024
OUT_FEATURES = 1024

BN = 1024

_LVL = [0, 1, 1, 2, 2, 2, 2, 3, 3, 3, 3, 3, 3, 3, 3]


def _fused_kernel(dims_ref, thr_ref, x_ref, lut_ref, out_ref, g_scr):
    @pl.when(pl.program_id(0) == 0)
    def _():
        gi = jax.lax.broadcasted_iota(jnp.int32, (IN_FEATURES, C * DEPTH), 0)
        g_scr[...] = jnp.where(gi == dims_ref[...], 1.0, 0.0).astype(jnp.bfloat16)

    chosen = jnp.dot(x_ref[...].astype(jnp.bfloat16), g_scr[...],
                     preferred_element_type=jnp.float32)
    chosenT = chosen.T

    b = []
    for j in range(15):
        v = chosenT[_LVL[j] * C:(_LVL[j] + 1) * C, :]
        b.append(jnp.where(v > thr_ref[:, j:j + 1], 1.0, 0.0))

    d0 = b[0]
    d1 = d0 * b[2] + (1.0 - d0) * b[1]
    d2l = d1 * b[4] + (1.0 - d1) * b[3]
    d2r = d1 * b[6] + (1.0 - d1) * b[5]
    d2 = d0 * d2r + (1.0 - d0) * d2l
    d3ll = d2 * b[8] + (1.0 - d2) * b[7]
    d3lr = d2 * b[10] + (1.0 - d2) * b[9]
    d3rl = d2 * b[12] + (1.0 - d2) * b[11]
    d3rr = d2 * b[14] + (1.0 - d2) * b[13]
    d3l = d1 * d3lr + (1.0 - d1) * d3ll
    d3r = d1 * d3rr + (1.0 - d1) * d3rl
    d3 = d0 * d3r + (1.0 - d0) * d3l
    idx = d0 * 8.0 + d1 * 4.0 + d2 * 2.0 + d3

    encT = jnp.concatenate(
        [jnp.where(idx == float(k), 1.0, 0.0) for k in range(K)], axis=0)
    out_ref[...] = jax.lax.dot_general(
        encT.astype(jnp.bfloat16), lut_ref[...],
        dimension_numbers=(((0,), (0,)), ((), ())),
        preferred_element_type=jnp.float32)


@jax.jit
def _run(x, dims, thresholds, lut):
    dims_lm = dims.astype(jnp.int32).reshape(C, DEPTH).T.reshape(1, C * DEPTH)
    thr_r = thresholds.reshape(C, K - 1)
    lutK = jnp.transpose(lut, (2, 1, 0)).reshape(K * C, OUT_FEATURES)
    lutK = lutK.astype(jnp.bfloat16)
    return pl.pallas_call(
        _fused_kernel,
        grid=(N_TOKENS // BN,),
        in_specs=[
            pl.BlockSpec((1, C * DEPTH), lambda i: (0, 0)),
            pl.BlockSpec((C, K - 1), lambda i: (0, 0)),
            pl.BlockSpec((BN, IN_FEATURES), lambda i: (i, 0)),
            pl.BlockSpec((K * C, OUT_FEATURES), lambda i: (0, 0)),
        ],
        out_specs=pl.BlockSpec((BN, OUT_FEATURES), lambda i: (i, 0)),
        out_shape=jax.ShapeDtypeStruct((N_TOKENS, OUT_FEATURES), jnp.float32),
        scratch_shapes=[pltpu.VMEM((IN_FEATURES, C * DEPTH), jnp.bfloat16)],
    )(dims_lm, thr_r, x, lutK)


def kernel(x, dims, thresholds, lut, S, T):
    return _run(x, dims, thresholds, lut)
